# baseline (speedup 1.0000x reference)
import jax
import jax.numpy as jnp
from jax import lax
from jax.experimental import pallas as pl
from jax.experimental.pallas import tpu as pltpu
from jax.experimental.pallas import tpu_sc as plsc

_N = 10000
_E = 320000
_H = 8
_D = 16
_DH = _H * _D
_NS = 16
_NW = 32
_C = 16
_SUB = 5
_EPT = _E // _NW
_NPT = 624
_NPTL = _N - 15 * _NPT
_NPAD = 10240
_NZ = _NPAD // 8
_NZT = _NZ // _NS
_NCH = _EPT // (_C * _SUB)


def _qkv_body(h_ref, wq_ref, wk_ref, wv_ref, b3_ref, q_ref, k_ref, v_ref):
    x = h_ref[...]
    q_ref[...] = jnp.dot(x, wq_ref[...], preferred_element_type=jnp.float32) + b3_ref[0:1, :]
    k_ref[...] = jnp.dot(x, wk_ref[...], preferred_element_type=jnp.float32) + b3_ref[1:2, :]
    v_ref[...] = jnp.dot(x, wv_ref[...], preferred_element_type=jnp.float32) + b3_ref[2:3, :]


def _qkv(h, wq, wk, wv, b3):
    out = jax.ShapeDtypeStruct((_N, _DH), jnp.float32)
    return pl.pallas_call(_qkv_body, out_shape=(out, out, out))(h, wq, wk, wv, b3)


_PE_BLK = 4000


def _pe_body(e_ref, w_ref, b_ref, o_ref):
    o_ref[...] = jnp.dot(e_ref[...], w_ref[...], preferred_element_type=jnp.float32) + b_ref[...]


def _proj_e(e, we, be2):
    grid = _E // _PE_BLK
    return pl.pallas_call(
        _pe_body,
        grid=(grid,),
        in_specs=[
            pl.BlockSpec((_PE_BLK, _DH), lambda i: (i, 0)),
            pl.BlockSpec((_DH, _DH), lambda i: (0, 0)),
            pl.BlockSpec((1, _DH), lambda i: (0, 0)),
        ],
        out_specs=pl.BlockSpec((_PE_BLK, _DH), lambda i: (i, 0)),
        out_shape=jax.ShapeDtypeStruct((_E, _DH), jnp.float32),
    )(e, we, be2)


def _lane_total(x, lanes):
    for sh in (8, 4, 2, 1):
        idx = lax.bitwise_xor(lanes, sh)
        x = x + x.at[idx].get(mode="promise_in_bounds")
    return x


def _acc_prologue(sid, v0):
    row0 = sid * _NPT
    nrows16 = jnp.where(sid == _NS - 1, _NPTL // 16, _NPT // 16)
    zrow0 = sid * _NZT

    def _zero_v(r, carry):
        for cix in range(8):
            v0[r, pl.ds(cix * 16, 16)] = jnp.zeros((16,), jnp.float32)
        return carry

    lax.fori_loop(0, _C, _zero_v, 0)
    return row0, nrows16, zrow0


def _acc_writeout(acc_wv, acc_z, wv_hbm, zp_hbm, row0, nrows16, zrow0, buf):
    def _wo(t, carry):
        pltpu.sync_copy(acc_wv.at[pl.ds(row0 + t * 16, 16)], buf)
        pltpu.sync_copy(buf, wv_hbm.at[pl.ds(row0 + t * 16, 16)])
        return carry

    lax.fori_loop(0, nrows16, _wo, 0)
    for jj in range(_NZT // 16):
        pltpu.sync_copy(acc_z.at[pl.ds(zrow0 + jj * 16, 16)], buf)
        pltpu.sync_copy(buf, zp_hbm.at[pl.ds(zrow0 + jj * 16, 16)])


def _make_store_z(didx, lanes):
    def _store_z(zb, i, j, srow_vec):
        dvec = didx[j, :]
        ilane = jnp.full((16,), i, jnp.int32)
        d7 = dvec.at[ilane].get(mode="promise_in_bounds") & 7
        one = jnp.full((16,), 1, jnp.int32)
        for cg in range(8):
            m = (one - jnp.minimum(lax.bitwise_xor(d7, cg), one)).astype(jnp.float32)
            zb[i, pl.ds(cg * 16, 16)] = srow_vec * m

    return _store_z


def _sparse_body(q_hbm, k_hbm, v_hbm, pe_hbm, src_hbm, dst_hbm,
                 eout_hbm, wv0_hbm, zp0_hbm, wv1_hbm, zp1_hbm,
                 sidx, didx, k0, k1, q0, v0, v1, z0, z1, perow, dz0, dz1,
                 acc_wv, acc_z,
                 semk0, semk1, semq, semv0, semv1,
                 semw0, semw1, semz0, semz1, semeo):
    core = lax.axis_index("c")
    sid = lax.axis_index("s")
    wid = sid * 2 + core
    lanes = lax.iota(jnp.int32, 16)
    kb_ = (k0, k1)
    vb_ = (v0, v1)
    zb_ = (z0, z1)
    dz_ = (dz0, dz1)
    semk_ = (semk0, semk1)
    semv_ = (semv0, semv1)
    semw_ = (semw0, semw1)
    semz_ = (semz0, semz1)
    _store_z = _make_store_z(didx, lanes)

    row0, nrows16, zrow0 = _acc_prologue(sid, v0)

    def _zero_acc(t, carry):
        pltpu.sync_copy(v0, acc_wv.at[pl.ds(row0 + t * 16, 16)])
        return carry

    lax.fori_loop(0, nrows16, _zero_acc, 0)
    for jj in range(_NZT // 16):
        pltpu.sync_copy(v0, acc_z.at[pl.ds(zrow0 + jj * 16, 16)])
    plsc.subcore_barrier()

    def _gathers(j, p):
        hk = pltpu.async_copy(k_hbm.at[sidx.at[j]], kb_[p], semk_[p])
        hv = pltpu.async_copy(v_hbm.at[sidx.at[j]], vb_[p], semv_[p])
        return hk, hv

    def _blk(c, carry):
        blk = wid * _NCH + c
        pltpu.sync_copy(src_hbm.at[blk], sidx)
        pltpu.sync_copy(dst_hbm.at[blk], didx)
        gath = {0: _gathers(0, 0)}
        hq = pltpu.async_copy(q_hbm.at[didx.at[0]], q0, semq)
        pend_wv = {}
        pend_z = {}
        pend_eo = {}
        for j in range(_SUB):
            p = j & 1
            kb, vb, zb = kb_[p], vb_[p], zb_[p]
            base = blk * (_C * _SUB) + j * _C
            if j >= 1:
                pend_eo[j - 1].wait()
            pltpu.sync_copy(pe_hbm.at[pl.ds(base, _C)], perow)
            if j + 1 < _SUB:
                if j >= 1:
                    pend_wv[j - 1].wait()
                    pend_z[j - 1].wait()
                gath[j + 1] = _gathers(j + 1, 1 - p)
            dz_[p][...] = lax.shift_right_logical(didx[j, :], 3)
            for hh in gath[j]:
                hh.wait()
            hq.wait()

            @plsc.parallel_loop(0, _C, unroll=2)
            def _edge(i):
                srow_vec = jnp.zeros((16,), jnp.float32)
                for hd in range(_H):
                    sl = pl.ds(hd * 16, 16)
                    eo = kb[i, sl] * q0[i, sl] * perow[i, sl]
                    kb[i, sl] = eo
                    sv = jnp.exp(jnp.clip(_lane_total(eo, lanes), -5.0, 5.0))
                    vb[i, sl] = vb[i, sl] * sv
                    srow_vec = jnp.where(lanes == hd, sv, srow_vec)
                _store_z(zb, i, j, srow_vec)

            if j + 1 < _SUB:
                hq = pltpu.async_copy(q_hbm.at[didx.at[j + 1]], q0, semq)
            pend_eo[j] = pltpu.async_copy(kb, eout_hbm.at[pl.ds(base, _C)], semeo)
            pend_wv[j] = pltpu.async_copy(vb, acc_wv.at[didx.at[j]], semw_[p],
                                          add=True)
            pend_z[j] = pltpu.async_copy(zb, acc_z.at[dz_[p]], semz_[p],
                                         add=True)
        pend_eo[_SUB - 1].wait()
        for j in (_SUB - 2, _SUB - 1):
            pend_wv[j].wait()
            pend_z[j].wait()
        return carry

    lax.fori_loop(0, _NCH, _blk, 0)
    plsc.subcore_barrier()

    @pl.when(core == 0)
    def _():
        _acc_writeout(acc_wv, acc_z, wv0_hbm, zp0_hbm, row0, nrows16, zrow0, v0)

    @pl.when(core == 1)
    def _():
        _acc_writeout(acc_wv, acc_z, wv1_hbm, zp1_hbm, row0, nrows16, zrow0, v0)


def _full_body(q_hbm, k_hbm, v_hbm, aux_hbm, src_hbm, dst_hbm,
               wv0_hbm, zp0_hbm, wv1_hbm, zp1_hbm,
               sidx, didx, k0, k1, q0, v0, v1, z0, z1, auxrow, dz0, dz1,
               acc_wv, acc_z,
               semk0, semk1, semq, semv0, semv1,
               semw0, semw1, semz0, semz1):
    core = lax.axis_index("c")
    sid = lax.axis_index("s")
    wid = sid * 2 + core
    lanes = lax.iota(jnp.int32, 16)
    kb_ = (k0, k1)
    vb_ = (v0, v1)
    zb_ = (z0, z1)
    dz_ = (dz0, dz1)
    semk_ = (semk0, semk1)
    semv_ = (semv0, semv1)
    semw_ = (semw0, semw1)
    semz_ = (semz0, semz1)
    _store_z = _make_store_z(didx, lanes)

    row0, nrows16, zrow0 = _acc_prologue(sid, v0)

    def _zero_acc(t, carry):
        pltpu.sync_copy(v0, acc_wv.at[pl.ds(row0 + t * 16, 16)])
        return carry

    lax.fori_loop(0, nrows16, _zero_acc, 0)
    for jj in range(_NZT // 16):
        pltpu.sync_copy(v0, acc_z.at[pl.ds(zrow0 + jj * 16, 16)])
    plsc.subcore_barrier()

    def _gathers(j, p):
        hk = pltpu.async_copy(k_hbm.at[sidx.at[j]], kb_[p], semk_[p])
        hv = pltpu.async_copy(v_hbm.at[sidx.at[j]], vb_[p], semv_[p])
        return hk, hv

    def _blk(c, carry):
        blk = wid * _NCH + c
        pltpu.sync_copy(src_hbm.at[blk], sidx)
        pltpu.sync_copy(dst_hbm.at[blk], didx)
        gath = {0: _gathers(0, 0)}
        hq = pltpu.async_copy(q_hbm.at[didx.at[0]], q0, semq)
        pend_wv = {}
        pend_z = {}
        for j in range(_SUB):
            p = j & 1
            kb, vb, zb = kb_[p], vb_[p], zb_[p]
            base = blk * (_C * _SUB) + j * _C
            pltpu.sync_copy(aux_hbm.at[pl.ds(base, _C)], auxrow)
            if j + 1 < _SUB:
                if j >= 1:
                    pend_wv[j - 1].wait()
                    pend_z[j - 1].wait()
                gath[j + 1] = _gathers(j + 1, 1 - p)
            dz_[p][...] = lax.shift_right_logical(didx[j, :], 3)
            for hh in gath[j]:
                hh.wait()
            hq.wait()

            @plsc.parallel_loop(0, _C, unroll=2)
            def _edge(i):
                srow_vec = jnp.zeros((16,), jnp.float32)
                auxvec = auxrow[i, :]
                av = auxvec[8]
                for hd in range(_H):
                    sl = pl.ds(hd * 16, 16)
                    tot = _lane_total(kb[i, sl] * q0[i, sl], lanes)
                    arg = jnp.clip(tot + auxvec[hd], -5.0, 5.0) * av
                    sv = jnp.exp(arg)
                    vb[i, sl] = vb[i, sl] * sv
                    srow_vec = jnp.where(lanes == hd, sv, srow_vec)
                _store_z(zb, i, j, srow_vec)

            if j + 1 < _SUB:
                hq = pltpu.async_copy(q_hbm.at[didx.at[j + 1]], q0, semq)
            pend_wv[j] = pltpu.async_copy(vb, acc_wv.at[didx.at[j]], semw_[p],
                                          add=True)
            pend_z[j] = pltpu.async_copy(zb, acc_z.at[dz_[p]], semz_[p],
                                         add=True)
        for j in (_SUB - 2, _SUB - 1):
            pend_wv[j].wait()
            pend_z[j].wait()
        return carry

    lax.fori_loop(0, _NCH, _blk, 0)
    plsc.subcore_barrier()

    @pl.when(core == 0)
    def _():
        _acc_writeout(acc_wv, acc_z, wv0_hbm, zp0_hbm, row0, nrows16, zrow0, v0)

    @pl.when(core == 1)
    def _():
        _acc_writeout(acc_wv, acc_z, wv1_hbm, zp1_hbm, row0, nrows16, zrow0, v0)


def _sc_scratch(f32, extra):
    return [
        pltpu.VMEM((_SUB, _C), jnp.int32),
        pltpu.VMEM((_SUB, _C), jnp.int32),
        pltpu.VMEM((_C, _DH), f32),
        pltpu.VMEM((_C, _DH), f32),
        pltpu.VMEM((_C, _DH), f32),
        pltpu.VMEM((_C, _DH), f32),
        pltpu.VMEM((_C, _DH), f32),
        pltpu.VMEM((_C, _DH), f32),
        pltpu.VMEM((_C, _DH), f32),
        extra,
        pltpu.VMEM((_C,), jnp.int32),
        pltpu.VMEM((_C,), jnp.int32),
        pltpu.VMEM_SHARED((_N, _DH), f32),
        pltpu.VMEM_SHARED((_NZ, _DH), f32),
    ]


def _sc_sparse(q, k, v, pe, src3, dst3):
    f32 = jnp.float32
    part_wv = jax.ShapeDtypeStruct((_N, _DH), f32)
    part_z = jax.ShapeDtypeStruct((_NZ, _DH), f32)
    fn = pl.kernel(
        _sparse_body,
        out_type=(jax.ShapeDtypeStruct((_E, _DH), f32),
                  part_wv, part_z, part_wv, part_z),
        mesh=plsc.VectorSubcoreMesh(core_axis_name="c", subcore_axis_name="s"),
        scratch_types=_sc_scratch(f32, pltpu.VMEM((_C, _DH), f32))
        + [pltpu.SemaphoreType.DMA] * 10,
    )
    return fn(q, k, v, pe, src3, dst3)


def _sc_full(q, k, v, aux, fsrc3, fdst3):
    f32 = jnp.float32
    part_wv = jax.ShapeDtypeStruct((_N, _DH), f32)
    part_z = jax.ShapeDtypeStruct((_NZ, _DH), f32)
    fn = pl.kernel(
        _full_body,
        out_type=(part_wv, part_z, part_wv, part_z),
        mesh=plsc.VectorSubcoreMesh(core_axis_name="c", subcore_axis_name="s"),
        scratch_types=_sc_scratch(f32, pltpu.VMEM((_C, 16), f32))
        + [pltpu.SemaphoreType.DMA] * 9,
    )
    return fn(q, k, v, aux, fsrc3, fdst3)


def _fin_body(wva0_ref, wva1_ref, za0_ref, za1_ref,
              wvb0_ref, wvb1_ref, zb0_ref, zb1_ref, m_ref, o_ref):
    m = m_ref[...]
    za = za0_ref[...][:, :_H] + za1_ref[...][:, :_H] + 1e-6
    zb = zb0_ref[...][:, :_H] + zb1_ref[...][:, :_H] + 1e-6
    zf = jnp.dot(za, m, preferred_element_type=jnp.float32)
    z2f = jnp.dot(zb, m, preferred_element_type=jnp.float32)
    o_ref[...] = ((wva0_ref[...] + wva1_ref[...]) / zf
                  + (wvb0_ref[...] + wvb1_ref[...]) / z2f)


def _finalize(wva0, wva1, za0, za1, wvb0, wvb1, zb0, zb1, m):
    return pl.pallas_call(
        _fin_body,
        out_shape=jax.ShapeDtypeStruct((_N, _DH), jnp.float32),
    )(wva0, wva1, za0, za1, wvb0, wvb1, zb0, zb1, m)


def kernel(h, e, edge_index, full_edge_index, adj2, rel_pos_3d,
           WQ, bQ, WK, bK, WV, bV, We, be):
    f32 = jnp.float32
    h = h.astype(f32)
    e = e.astype(f32)
    src3 = edge_index[0].astype(jnp.int32).reshape(-1, _SUB, _C)
    dst3 = edge_index[1].astype(jnp.int32).reshape(-1, _SUB, _C)
    fsrc3 = full_edge_index[0].astype(jnp.int32).reshape(-1, _SUB, _C)
    fdst3 = full_edge_index[1].astype(jnp.int32).reshape(-1, _SUB, _C)
    aux = jnp.concatenate(
        [rel_pos_3d.astype(f32), adj2.reshape(-1, 1).astype(f32),
         jnp.zeros((_E, 7), f32)], axis=1)

    wq = (WQ * 0.25).astype(f32)
    b3 = jnp.stack([bQ * 0.25, bK, bV]).astype(f32)

    q, k, v = _qkv(h, wq, WK.astype(f32), WV.astype(f32), b3)
    pe = _proj_e(e, We.astype(f32), be.reshape(1, _DH).astype(f32))

    eout, wva0, zpa0, wva1, zpa1 = _sc_sparse(q, k, v, pe, src3, dst3)
    wvb0, zpb0, wvb1, zpb1 = _sc_full(q, k, v, aux, fsrc3, fdst3)

    za0 = zpa0.reshape(_NPAD, 16)[:_N]
    za1 = zpa1.reshape(_NPAD, 16)[:_N]
    zb0 = zpb0.reshape(_NPAD, 16)[:_N]
    zb1 = zpb1.reshape(_NPAD, 16)[:_N]

    m = (jnp.arange(_DH)[None, :] // _D == jnp.arange(_H)[:, None]).astype(f32)
    h_out = _finalize(wva0, wva1, za0, za1, wvb0, wvb1, zb0, zb1, m)

    return h_out.reshape(_N, _H, _D), eout.reshape(_E, _H, _D)


# speedup vs baseline: 1.1557x; 1.1557x over previous
import jax
import jax.numpy as jnp
from jax import lax
from jax.experimental import pallas as pl
from jax.experimental.pallas import tpu as pltpu
from jax.experimental.pallas import tpu_sc as plsc

_N = 10000
_E = 320000
_H = 8
_D = 16
_DH = _H * _D
_NS = 16
_NW = 32
_C = 16
_SUB = 5
_EPT = _E // _NW
_NPT = 624
_NPTL = _N - 15 * _NPT
_NPAD = 10240
_NZ = _NPAD // 8
_NZT = _NZ // _NS
_NCH = _EPT // (_C * _SUB)


def _qkv_body(h_ref, wq_ref, wk_ref, wv_ref, b3_ref, q_ref, k_ref, v_ref):
    x = h_ref[...]
    q_ref[...] = jnp.dot(x, wq_ref[...], preferred_element_type=jnp.float32) + b3_ref[0:1, :]
    k_ref[...] = jnp.dot(x, wk_ref[...], preferred_element_type=jnp.float32) + b3_ref[1:2, :]
    v_ref[...] = jnp.dot(x, wv_ref[...], preferred_element_type=jnp.float32) + b3_ref[2:3, :]


def _qkv(h, wq, wk, wv, b3):
    out = jax.ShapeDtypeStruct((_N, _DH), jnp.float32)
    return pl.pallas_call(_qkv_body, out_shape=(out, out, out))(h, wq, wk, wv, b3)


_PE_BLK = 4000


def _pe_body(e_ref, w_ref, b_ref, o_ref):
    o_ref[...] = jnp.dot(e_ref[...], w_ref[...], preferred_element_type=jnp.float32) + b_ref[...]


def _proj_e(e, we, be2):
    grid = _E // _PE_BLK
    return pl.pallas_call(
        _pe_body,
        grid=(grid,),
        in_specs=[
            pl.BlockSpec((_PE_BLK, _DH), lambda i: (i, 0)),
            pl.BlockSpec((_DH, _DH), lambda i: (0, 0)),
            pl.BlockSpec((1, _DH), lambda i: (0, 0)),
        ],
        out_specs=pl.BlockSpec((_PE_BLK, _DH), lambda i: (i, 0)),
        out_shape=jax.ShapeDtypeStruct((_E, _DH), jnp.float32),
    )(e, we, be2)


def _lane_total(x, lanes):
    for sh in (8, 4, 2, 1):
        idx = lax.bitwise_xor(lanes, sh)
        x = x + x.at[idx].get(mode="promise_in_bounds")
    return x


def _acc_prologue(sid, v0):
    row0 = sid * _NPT
    nrows16 = jnp.where(sid == _NS - 1, _NPTL // 16, _NPT // 16)
    zrow0 = sid * _NZT

    def _zero_v(r, carry):
        for cix in range(8):
            v0[r, pl.ds(cix * 16, 16)] = jnp.zeros((16,), jnp.float32)
        return carry

    lax.fori_loop(0, _C, _zero_v, 0)
    return row0, nrows16, zrow0


def _acc_writeout(acc_wv, acc_z, wv_hbm, zp_hbm, row0, nrows16, zrow0, buf):
    def _wo(t, carry):
        pltpu.sync_copy(acc_wv.at[pl.ds(row0 + t * 16, 16)], buf)
        pltpu.sync_copy(buf, wv_hbm.at[pl.ds(row0 + t * 16, 16)])
        return carry

    lax.fori_loop(0, nrows16, _wo, 0)
    for jj in range(_NZT // 16):
        pltpu.sync_copy(acc_z.at[pl.ds(zrow0 + jj * 16, 16)], buf)
        pltpu.sync_copy(buf, zp_hbm.at[pl.ds(zrow0 + jj * 16, 16)])


def _make_store_z(didx, lanes):
    def _store_z(zb, i, j, srow_vec):
        dvec = didx[j, :]
        ilane = jnp.full((16,), i, jnp.int32)
        d7 = dvec.at[ilane].get(mode="promise_in_bounds") & 7
        one = jnp.full((16,), 1, jnp.int32)
        for cg in range(8):
            m = (one - jnp.minimum(lax.bitwise_xor(d7, cg), one)).astype(jnp.float32)
            zb[i, pl.ds(cg * 16, 16)] = srow_vec * m

    return _store_z


def _sparse_body(q_hbm, k_hbm, v_hbm, pe_hbm, src_hbm, dst_hbm,
                 eout_hbm, wv0_hbm, zp0_hbm, wv1_hbm, zp1_hbm,
                 sidx, didx, k0, k1, q0, v0, v1, z0, z1, perow, dz0, dz1,
                 acc_wv, acc_z,
                 semk0, semk1, semq, semv0, semv1,
                 semw0, semw1, semz0, semz1, semeo):
    core = lax.axis_index("c")
    sid = lax.axis_index("s")
    wid = sid * 2 + core
    lanes = lax.iota(jnp.int32, 16)
    kb_ = (k0, k1)
    vb_ = (v0, v1)
    zb_ = (z0, z1)
    dz_ = (dz0, dz1)
    semk_ = (semk0, semk1)
    semv_ = (semv0, semv1)
    semw_ = (semw0, semw1)
    semz_ = (semz0, semz1)
    _store_z = _make_store_z(didx, lanes)

    row0, nrows16, zrow0 = _acc_prologue(sid, v0)

    def _zero_acc(t, carry):
        pltpu.sync_copy(v0, acc_wv.at[pl.ds(row0 + t * 16, 16)])
        return carry

    lax.fori_loop(0, nrows16, _zero_acc, 0)
    for jj in range(_NZT // 16):
        pltpu.sync_copy(v0, acc_z.at[pl.ds(zrow0 + jj * 16, 16)])
    plsc.subcore_barrier()

    def _gathers(j, p):
        hk = pltpu.async_copy(k_hbm.at[sidx.at[j]], kb_[p], semk_[p])
        hv = pltpu.async_copy(v_hbm.at[sidx.at[j]], vb_[p], semv_[p])
        return hk, hv

    def _blk(c, carry):
        blk = wid * _NCH + c
        pltpu.sync_copy(src_hbm.at[blk], sidx)
        pltpu.sync_copy(dst_hbm.at[blk], didx)
        gath = {0: _gathers(0, 0)}
        hq = pltpu.async_copy(q_hbm.at[didx.at[0]], q0, semq)
        pend_wv = {}
        pend_z = {}
        pend_eo = {}
        for j in range(_SUB):
            p = j & 1
            kb, vb, zb = kb_[p], vb_[p], zb_[p]
            base = blk * (_C * _SUB) + j * _C
            if j >= 1:
                pend_eo[j - 1].wait()
            pltpu.sync_copy(pe_hbm.at[pl.ds(base, _C)], perow)
            if j + 1 < _SUB:
                if j >= 1:
                    pend_wv[j - 1].wait()
                    pend_z[j - 1].wait()
                gath[j + 1] = _gathers(j + 1, 1 - p)
            dz_[p][...] = lax.shift_right_logical(didx[j, :], 3)
            for hh in gath[j]:
                hh.wait()
            hq.wait()

            @plsc.parallel_loop(0, _C, unroll=1)
            def _edge(i):
                srow_vec = jnp.zeros((16,), jnp.float32)
                for hd in range(_H):
                    sl = pl.ds(hd * 16, 16)
                    eo = kb[i, sl] * q0[i, sl] * perow[i, sl]
                    kb[i, sl] = eo
                    sv = jnp.exp(jnp.clip(_lane_total(eo, lanes), -5.0, 5.0))
                    vb[i, sl] = vb[i, sl] * sv
                    srow_vec = jnp.where(lanes == hd, sv, srow_vec)
                _store_z(zb, i, j, srow_vec)

            if j + 1 < _SUB:
                hq = pltpu.async_copy(q_hbm.at[didx.at[j + 1]], q0, semq)
            pend_eo[j] = pltpu.async_copy(kb, eout_hbm.at[pl.ds(base, _C)], semeo)
            pend_wv[j] = pltpu.async_copy(vb, acc_wv.at[didx.at[j]], semw_[p],
                                          add=True)
            pend_z[j] = pltpu.async_copy(zb, acc_z.at[dz_[p]], semz_[p],
                                         add=True)
        pend_eo[_SUB - 1].wait()
        for j in (_SUB - 2, _SUB - 1):
            pend_wv[j].wait()
            pend_z[j].wait()
        return carry

    lax.fori_loop(0, _NCH, _blk, 0)
    plsc.subcore_barrier()

    @pl.when(core == 0)
    def _():
        _acc_writeout(acc_wv, acc_z, wv0_hbm, zp0_hbm, row0, nrows16, zrow0, v0)

    @pl.when(core == 1)
    def _():
        _acc_writeout(acc_wv, acc_z, wv1_hbm, zp1_hbm, row0, nrows16, zrow0, v0)


def _full_body(q_hbm, k_hbm, v_hbm, aux_hbm, src_hbm, dst_hbm,
               wv0_hbm, zp0_hbm, wv1_hbm, zp1_hbm,
               sidx, didx, k0, k1, q0, v0, v1, z0, z1, auxrow, dz0, dz1,
               acc_wv, acc_z,
               semk0, semk1, semq, semv0, semv1,
               semw0, semw1, semz0, semz1):
    core = lax.axis_index("c")
    sid = lax.axis_index("s")
    wid = sid * 2 + core
    lanes = lax.iota(jnp.int32, 16)
    kb_ = (k0, k1)
    vb_ = (v0, v1)
    zb_ = (z0, z1)
    dz_ = (dz0, dz1)
    semk_ = (semk0, semk1)
    semv_ = (semv0, semv1)
    semw_ = (semw0, semw1)
    semz_ = (semz0, semz1)
    _store_z = _make_store_z(didx, lanes)

    row0, nrows16, zrow0 = _acc_prologue(sid, v0)

    def _zero_acc(t, carry):
        pltpu.sync_copy(v0, acc_wv.at[pl.ds(row0 + t * 16, 16)])
        return carry

    lax.fori_loop(0, nrows16, _zero_acc, 0)
    for jj in range(_NZT // 16):
        pltpu.sync_copy(v0, acc_z.at[pl.ds(zrow0 + jj * 16, 16)])
    plsc.subcore_barrier()

    def _gathers(j, p):
        hk = pltpu.async_copy(k_hbm.at[sidx.at[j]], kb_[p], semk_[p])
        hv = pltpu.async_copy(v_hbm.at[sidx.at[j]], vb_[p], semv_[p])
        return hk, hv

    def _blk(c, carry):
        blk = wid * _NCH + c
        pltpu.sync_copy(src_hbm.at[blk], sidx)
        pltpu.sync_copy(dst_hbm.at[blk], didx)
        gath = {0: _gathers(0, 0)}
        hq = pltpu.async_copy(q_hbm.at[didx.at[0]], q0, semq)
        pend_wv = {}
        pend_z = {}
        for j in range(_SUB):
            p = j & 1
            kb, vb, zb = kb_[p], vb_[p], zb_[p]
            base = blk * (_C * _SUB) + j * _C
            pltpu.sync_copy(aux_hbm.at[pl.ds(base, _C)], auxrow)
            if j + 1 < _SUB:
                if j >= 1:
                    pend_wv[j - 1].wait()
                    pend_z[j - 1].wait()
                gath[j + 1] = _gathers(j + 1, 1 - p)
            dz_[p][...] = lax.shift_right_logical(didx[j, :], 3)
            for hh in gath[j]:
                hh.wait()
            hq.wait()

            @plsc.parallel_loop(0, _C, unroll=1)
            def _edge(i):
                srow_vec = jnp.zeros((16,), jnp.float32)
                auxvec = auxrow[i, :]
                av = auxvec[8]
                for hd in range(_H):
                    sl = pl.ds(hd * 16, 16)
                    tot = _lane_total(kb[i, sl] * q0[i, sl], lanes)
                    arg = jnp.clip(tot + auxvec[hd], -5.0, 5.0) * av
                    sv = jnp.exp(arg)
                    vb[i, sl] = vb[i, sl] * sv
                    srow_vec = jnp.where(lanes == hd, sv, srow_vec)
                _store_z(zb, i, j, srow_vec)

            if j + 1 < _SUB:
                hq = pltpu.async_copy(q_hbm.at[didx.at[j + 1]], q0, semq)
            pend_wv[j] = pltpu.async_copy(vb, acc_wv.at[didx.at[j]], semw_[p],
                                          add=True)
            pend_z[j] = pltpu.async_copy(zb, acc_z.at[dz_[p]], semz_[p],
                                         add=True)
        for j in (_SUB - 2, _SUB - 1):
            pend_wv[j].wait()
            pend_z[j].wait()
        return carry

    lax.fori_loop(0, _NCH, _blk, 0)
    plsc.subcore_barrier()

    @pl.when(core == 0)
    def _():
        _acc_writeout(acc_wv, acc_z, wv0_hbm, zp0_hbm, row0, nrows16, zrow0, v0)

    @pl.when(core == 1)
    def _():
        _acc_writeout(acc_wv, acc_z, wv1_hbm, zp1_hbm, row0, nrows16, zrow0, v0)


def _sc_scratch(f32, extra):
    return [
        pltpu.VMEM((_SUB, _C), jnp.int32),
        pltpu.VMEM((_SUB, _C), jnp.int32),
        pltpu.VMEM((_C, _DH), f32),
        pltpu.VMEM((_C, _DH), f32),
        pltpu.VMEM((_C, _DH), f32),
        pltpu.VMEM((_C, _DH), f32),
        pltpu.VMEM((_C, _DH), f32),
        pltpu.VMEM((_C, _DH), f32),
        pltpu.VMEM((_C, _DH), f32),
        extra,
        pltpu.VMEM((_C,), jnp.int32),
        pltpu.VMEM((_C,), jnp.int32),
        pltpu.VMEM_SHARED((_N, _DH), f32),
        pltpu.VMEM_SHARED((_NZ, _DH), f32),
    ]


def _sc_sparse(q, k, v, pe, src3, dst3):
    f32 = jnp.float32
    part_wv = jax.ShapeDtypeStruct((_N, _DH), f32)
    part_z = jax.ShapeDtypeStruct((_NZ, _DH), f32)
    fn = pl.kernel(
        _sparse_body,
        out_type=(jax.ShapeDtypeStruct((_E, _DH), f32),
                  part_wv, part_z, part_wv, part_z),
        mesh=plsc.VectorSubcoreMesh(core_axis_name="c", subcore_axis_name="s"),
        scratch_types=_sc_scratch(f32, pltpu.VMEM((_C, _DH), f32))
        + [pltpu.SemaphoreType.DMA] * 10,
    )
    return fn(q, k, v, pe, src3, dst3)


def _sc_full(q, k, v, aux, fsrc3, fdst3):
    f32 = jnp.float32
    part_wv = jax.ShapeDtypeStruct((_N, _DH), f32)
    part_z = jax.ShapeDtypeStruct((_NZ, _DH), f32)
    fn = pl.kernel(
        _full_body,
        out_type=(part_wv, part_z, part_wv, part_z),
        mesh=plsc.VectorSubcoreMesh(core_axis_name="c", subcore_axis_name="s"),
        scratch_types=_sc_scratch(f32, pltpu.VMEM((_C, 16), f32))
        + [pltpu.SemaphoreType.DMA] * 9,
    )
    return fn(q, k, v, aux, fsrc3, fdst3)


def _fin_body(wva0_ref, wva1_ref, za0_ref, za1_ref,
              wvb0_ref, wvb1_ref, zb0_ref, zb1_ref, m_ref, o_ref):
    m = m_ref[...]
    za = za0_ref[...][:, :_H] + za1_ref[...][:, :_H] + 1e-6
    zb = zb0_ref[...][:, :_H] + zb1_ref[...][:, :_H] + 1e-6
    zf = jnp.dot(za, m, preferred_element_type=jnp.float32)
    z2f = jnp.dot(zb, m, preferred_element_type=jnp.float32)
    o_ref[...] = ((wva0_ref[...] + wva1_ref[...]) / zf
                  + (wvb0_ref[...] + wvb1_ref[...]) / z2f)


def _finalize(wva0, wva1, za0, za1, wvb0, wvb1, zb0, zb1, m):
    return pl.pallas_call(
        _fin_body,
        out_shape=jax.ShapeDtypeStruct((_N, _DH), jnp.float32),
    )(wva0, wva1, za0, za1, wvb0, wvb1, zb0, zb1, m)


def kernel(h, e, edge_index, full_edge_index, adj2, rel_pos_3d,
           WQ, bQ, WK, bK, WV, bV, We, be):
    f32 = jnp.float32
    h = h.astype(f32)
    e = e.astype(f32)
    src3 = edge_index[0].astype(jnp.int32).reshape(-1, _SUB, _C)
    dst3 = edge_index[1].astype(jnp.int32).reshape(-1, _SUB, _C)
    fsrc3 = full_edge_index[0].astype(jnp.int32).reshape(-1, _SUB, _C)
    fdst3 = full_edge_index[1].astype(jnp.int32).reshape(-1, _SUB, _C)
    aux = jnp.concatenate(
        [rel_pos_3d.astype(f32), adj2.reshape(-1, 1).astype(f32),
         jnp.zeros((_E, 7), f32)], axis=1)

    wq = (WQ * 0.25).astype(f32)
    b3 = jnp.stack([bQ * 0.25, bK, bV]).astype(f32)

    q, k, v = _qkv(h, wq, WK.astype(f32), WV.astype(f32), b3)
    pe = _proj_e(e, We.astype(f32), be.reshape(1, _DH).astype(f32))

    eout, wva0, zpa0, wva1, zpa1 = _sc_sparse(q, k, v, pe, src3, dst3)
    wvb0, zpb0, wvb1, zpb1 = _sc_full(q, k, v, aux, fsrc3, fdst3)

    za0 = zpa0.reshape(_NPAD, 16)[:_N]
    za1 = zpa1.reshape(_NPAD, 16)[:_N]
    zb0 = zpb0.reshape(_NPAD, 16)[:_N]
    zb1 = zpb1.reshape(_NPAD, 16)[:_N]

    m = (jnp.arange(_DH)[None, :] // _D == jnp.arange(_H)[:, None]).astype(f32)
    h_out = _finalize(wva0, wva1, za0, za1, wvb0, wvb1, zb0, zb1, m)

    return h_out.reshape(_N, _H, _D), eout.reshape(_E, _H, _D)


# speedup vs baseline: 1.1558x; 1.0001x over previous
import jax
import jax.numpy as jnp
from jax import lax
from jax.experimental import pallas as pl
from jax.experimental.pallas import tpu as pltpu
from jax.experimental.pallas import tpu_sc as plsc

_N = 10000
_E = 320000
_H = 8
_D = 16
_DH = _H * _D
_NS = 16
_NW = 32
_C = 16
_SUB = 5
_EPT = _E // _NW
_NPT = 624
_NPTL = _N - 15 * _NPT
_NPAD = 10240
_NZ = _NPAD // 8
_NZT = _NZ // _NS
_NCH = _EPT // (_C * _SUB)


def _qkv_body(h_ref, wq_ref, wk_ref, wv_ref, b3_ref, q_ref, k_ref, v_ref):
    x = h_ref[...]
    q_ref[...] = jnp.dot(x, wq_ref[...], preferred_element_type=jnp.float32) + b3_ref[0:1, :]
    k_ref[...] = jnp.dot(x, wk_ref[...], preferred_element_type=jnp.float32) + b3_ref[1:2, :]
    v_ref[...] = jnp.dot(x, wv_ref[...], preferred_element_type=jnp.float32) + b3_ref[2:3, :]


def _qkv(h, wq, wk, wv, b3):
    out = jax.ShapeDtypeStruct((_N, _DH), jnp.float32)
    return pl.pallas_call(_qkv_body, out_shape=(out, out, out))(h, wq, wk, wv, b3)


_PE_BLK = 4000


def _pe_body(e_ref, w_ref, b_ref, o_ref):
    o_ref[...] = jnp.dot(e_ref[...], w_ref[...], preferred_element_type=jnp.float32) + b_ref[...]


def _proj_e(e, we, be2):
    grid = _E // _PE_BLK
    return pl.pallas_call(
        _pe_body,
        grid=(grid,),
        in_specs=[
            pl.BlockSpec((_PE_BLK, _DH), lambda i: (i, 0)),
            pl.BlockSpec((_DH, _DH), lambda i: (0, 0)),
            pl.BlockSpec((1, _DH), lambda i: (0, 0)),
        ],
        out_specs=pl.BlockSpec((_PE_BLK, _DH), lambda i: (i, 0)),
        out_shape=jax.ShapeDtypeStruct((_E, _DH), jnp.float32),
    )(e, we, be2)


def _lane_total(x, lanes):
    for sh in (8, 4, 2, 1):
        idx = lax.bitwise_xor(lanes, sh)
        x = x + x.at[idx].get(mode="promise_in_bounds")
    return x


def _acc_prologue(sid, v0):
    row0 = sid * _NPT
    nrows16 = jnp.where(sid == _NS - 1, _NPTL // 16, _NPT // 16)
    zrow0 = sid * _NZT

    def _zero_v(r, carry):
        for cix in range(8):
            v0[r, pl.ds(cix * 16, 16)] = jnp.zeros((16,), jnp.float32)
        return carry

    lax.fori_loop(0, _C, _zero_v, 0)
    return row0, nrows16, zrow0


def _acc_writeout(acc_wv, acc_z, wv_hbm, zp_hbm, row0, nrows16, zrow0, buf):
    def _wo(t, carry):
        pltpu.sync_copy(acc_wv.at[pl.ds(row0 + t * 16, 16)], buf)
        pltpu.sync_copy(buf, wv_hbm.at[pl.ds(row0 + t * 16, 16)])
        return carry

    lax.fori_loop(0, nrows16, _wo, 0)
    for jj in range(_NZT // 16):
        pltpu.sync_copy(acc_z.at[pl.ds(zrow0 + jj * 16, 16)], buf)
        pltpu.sync_copy(buf, zp_hbm.at[pl.ds(zrow0 + jj * 16, 16)])


def _make_store_z(didx, lanes):
    def _store_z(zb, i, j, srow_vec):
        dvec = didx[j, :]
        ilane = jnp.full((16,), i, jnp.int32)
        d7 = dvec.at[ilane].get(mode="promise_in_bounds") & 7
        one = jnp.full((16,), 1, jnp.int32)
        for cg in range(8):
            m = (one - jnp.minimum(lax.bitwise_xor(d7, cg), one)).astype(jnp.float32)
            zb[i, pl.ds(cg * 16, 16)] = srow_vec * m

    return _store_z


def _sparse_body(q_hbm, k_hbm, v_hbm, pe_hbm, src_hbm, dst_hbm,
                 eout_hbm, wv0_hbm, zp0_hbm, wv1_hbm, zp1_hbm,
                 sidx, didx, k0, k1, q0, v0, v1, z0, z1, perow, dz0, dz1,
                 acc_wv, acc_z,
                 semk0, semk1, semq, semv0, semv1,
                 semw0, semw1, semz0, semz1, semeo):
    core = lax.axis_index("c")
    sid = lax.axis_index("s")
    wid = sid * 2 + core
    lanes = lax.iota(jnp.int32, 16)
    kb_ = (k0, k1)
    vb_ = (v0, v1)
    zb_ = (z0, z1)
    dz_ = (dz0, dz1)
    semk_ = (semk0, semk1)
    semv_ = (semv0, semv1)
    semw_ = (semw0, semw1)
    semz_ = (semz0, semz1)
    _store_z = _make_store_z(didx, lanes)

    row0, nrows16, zrow0 = _acc_prologue(sid, v0)

    def _zero_acc(t, carry):
        pltpu.sync_copy(v0, acc_wv.at[pl.ds(row0 + t * 16, 16)])
        return carry

    lax.fori_loop(0, nrows16, _zero_acc, 0)
    for jj in range(_NZT // 16):
        pltpu.sync_copy(v0, acc_z.at[pl.ds(zrow0 + jj * 16, 16)])
    plsc.subcore_barrier()

    def _gathers(j, p):
        hk = pltpu.async_copy(k_hbm.at[sidx.at[j]], kb_[p], semk_[p])
        hv = pltpu.async_copy(v_hbm.at[sidx.at[j]], vb_[p], semv_[p])
        return hk, hv

    def _blk(c, carry):
        blk = wid * _NCH + c
        pltpu.sync_copy(src_hbm.at[blk], sidx)
        pltpu.sync_copy(dst_hbm.at[blk], didx)
        gath = {0: _gathers(0, 0)}
        hq = pltpu.async_copy(q_hbm.at[didx.at[0]], q0, semq)
        pend_wv = {}
        pend_z = {}
        pend_eo = {}
        for j in range(_SUB):
            p = j & 1
            kb, vb, zb = kb_[p], vb_[p], zb_[p]
            base = blk * (_C * _SUB) + j * _C
            if j >= 1:
                pend_eo[j - 1].wait()
            pltpu.sync_copy(pe_hbm.at[pl.ds(base, _C)], perow)
            if j + 1 < _SUB:
                if j >= 1:
                    pend_wv[j - 1].wait()
                    pend_z[j - 1].wait()
                gath[j + 1] = _gathers(j + 1, 1 - p)
            dz_[p][...] = lax.shift_right_logical(didx[j, :], 3)
            for hh in gath[j]:
                hh.wait()
            hq.wait()

            @plsc.parallel_loop(0, _C, unroll=1)
            def _edge(i):
                srow_vec = jnp.zeros((16,), jnp.float32)
                for hd in range(_H):
                    sl = pl.ds(hd * 16, 16)
                    eo = kb[i, sl] * q0[i, sl] * perow[i, sl]
                    kb[i, sl] = eo
                    sv = jnp.exp(jnp.clip(_lane_total(eo, lanes), -5.0, 5.0))
                    vb[i, sl] = vb[i, sl] * sv
                    srow_vec = jnp.where(lanes == hd, sv, srow_vec)
                _store_z(zb, i, j, srow_vec)

            if j + 1 < _SUB:
                hq = pltpu.async_copy(q_hbm.at[didx.at[j + 1]], q0, semq)
            pend_eo[j] = pltpu.async_copy(kb, eout_hbm.at[pl.ds(base, _C)], semeo)
            pend_wv[j] = pltpu.async_copy(vb, acc_wv.at[didx.at[j]], semw_[p],
                                          add=True)
            pend_z[j] = pltpu.async_copy(zb, acc_z.at[dz_[p]], semz_[p],
                                         add=True)
        pend_eo[_SUB - 1].wait()
        for j in (_SUB - 2, _SUB - 1):
            pend_wv[j].wait()
            pend_z[j].wait()
        return carry

    lax.fori_loop(0, _NCH, _blk, 0)
    plsc.subcore_barrier()

    @pl.when(core == 0)
    def _():
        _acc_writeout(acc_wv, acc_z, wv0_hbm, zp0_hbm, row0, nrows16, zrow0, v0)

    @pl.when(core == 1)
    def _():
        _acc_writeout(acc_wv, acc_z, wv1_hbm, zp1_hbm, row0, nrows16, zrow0, v0)


def _full_body(q_hbm, k_hbm, v_hbm, aux_hbm, src_hbm, dst_hbm,
               wv0_hbm, zp0_hbm, wv1_hbm, zp1_hbm,
               sidx, didx, k0, k1, q0, v0, v1, z0, z1, auxrow, dz0, dz1,
               acc_wv, acc_z,
               semk0, semk1, semq, semv0, semv1,
               semw0, semw1, semz0, semz1):
    core = lax.axis_index("c")
    sid = lax.axis_index("s")
    wid = sid * 2 + core
    lanes = lax.iota(jnp.int32, 16)
    kb_ = (k0, k1)
    vb_ = (v0, v1)
    zb_ = (z0, z1)
    dz_ = (dz0, dz1)
    semk_ = (semk0, semk1)
    semv_ = (semv0, semv1)
    semw_ = (semw0, semw1)
    semz_ = (semz0, semz1)
    _store_z = _make_store_z(didx, lanes)

    row0, nrows16, zrow0 = _acc_prologue(sid, v0)

    def _zero_acc(t, carry):
        pltpu.sync_copy(v0, acc_wv.at[pl.ds(row0 + t * 16, 16)])
        return carry

    lax.fori_loop(0, nrows16, _zero_acc, 0)
    for jj in range(_NZT // 16):
        pltpu.sync_copy(v0, acc_z.at[pl.ds(zrow0 + jj * 16, 16)])
    plsc.subcore_barrier()

    def _gathers(j, p):
        hk = pltpu.async_copy(k_hbm.at[sidx.at[j]], kb_[p], semk_[p])
        hv = pltpu.async_copy(v_hbm.at[sidx.at[j]], vb_[p], semv_[p])
        return hk, hv

    def _blk(c, carry):
        blk = wid * _NCH + c
        pltpu.sync_copy(src_hbm.at[blk], sidx)
        pltpu.sync_copy(dst_hbm.at[blk], didx)
        gath = {0: _gathers(0, 0)}
        hq = pltpu.async_copy(q_hbm.at[didx.at[0]], q0, semq)
        pend_wv = {}
        pend_z = {}
        for j in range(_SUB):
            p = j & 1
            kb, vb, zb = kb_[p], vb_[p], zb_[p]
            base = blk * (_C * _SUB) + j * _C
            pltpu.sync_copy(aux_hbm.at[pl.ds(base, _C)], auxrow)
            if j + 1 < _SUB:
                if j >= 1:
                    pend_wv[j - 1].wait()
                    pend_z[j - 1].wait()
                gath[j + 1] = _gathers(j + 1, 1 - p)
            dz_[p][...] = lax.shift_right_logical(didx[j, :], 3)
            for hh in gath[j]:
                hh.wait()
            hq.wait()

            @plsc.parallel_loop(0, _C, unroll=1)
            def _edge(i):
                srow_vec = jnp.zeros((16,), jnp.float32)
                auxvec = auxrow[i, :]
                av = auxvec[8]
                for hd in range(_H):
                    sl = pl.ds(hd * 16, 16)
                    tot = _lane_total(kb[i, sl] * q0[i, sl], lanes)
                    arg = jnp.clip(tot + auxvec[hd], -5.0, 5.0) * av
                    sv = jnp.exp(arg)
                    vb[i, sl] = vb[i, sl] * sv
                    srow_vec = jnp.where(lanes == hd, sv, srow_vec)
                _store_z(zb, i, j, srow_vec)

            if j + 1 < _SUB:
                hq = pltpu.async_copy(q_hbm.at[didx.at[j + 1]], q0, semq)
            pend_wv[j] = pltpu.async_copy(vb, acc_wv.at[didx.at[j]], semw_[p],
                                          add=True)
            pend_z[j] = pltpu.async_copy(zb, acc_z.at[dz_[p]], semz_[p],
                                         add=True)
        for j in (_SUB - 2, _SUB - 1):
            pend_wv[j].wait()
            pend_z[j].wait()
        return carry

    lax.fori_loop(0, _NCH, _blk, 0)
    plsc.subcore_barrier()

    @pl.when(core == 0)
    def _():
        _acc_writeout(acc_wv, acc_z, wv0_hbm, zp0_hbm, row0, nrows16, zrow0, v0)

    @pl.when(core == 1)
    def _():
        _acc_writeout(acc_wv, acc_z, wv1_hbm, zp1_hbm, row0, nrows16, zrow0, v0)


def _sc_scratch(f32, extra):
    return [
        pltpu.VMEM((_SUB, _C), jnp.int32),
        pltpu.VMEM((_SUB, _C), jnp.int32),
        pltpu.VMEM((_C, _DH), f32),
        pltpu.VMEM((_C, _DH), f32),
        pltpu.VMEM((_C, _DH), f32),
        pltpu.VMEM((_C, _DH), f32),
        pltpu.VMEM((_C, _DH), f32),
        pltpu.VMEM((_C, _DH), f32),
        pltpu.VMEM((_C, _DH), f32),
        extra,
        pltpu.VMEM((_C,), jnp.int32),
        pltpu.VMEM((_C,), jnp.int32),
        pltpu.VMEM_SHARED((_N, _DH), f32),
        pltpu.VMEM_SHARED((_NZ, _DH), f32),
    ]


def _sc_sparse(q, k, v, pe, src3, dst3):
    f32 = jnp.float32
    part_wv = jax.ShapeDtypeStruct((_N, _DH), f32)
    part_z = jax.ShapeDtypeStruct((_NZ, _DH), f32)
    fn = pl.kernel(
        _sparse_body,
        out_type=(jax.ShapeDtypeStruct((_E, _DH), f32),
                  part_wv, part_z, part_wv, part_z),
        mesh=plsc.VectorSubcoreMesh(core_axis_name="c", subcore_axis_name="s"),
        scratch_types=_sc_scratch(f32, pltpu.VMEM((_C, _DH), f32))
        + [pltpu.SemaphoreType.DMA] * 10,
    )
    return fn(q, k, v, pe, src3, dst3)


def _sc_full(q, k, v, aux, fsrc3, fdst3):
    f32 = jnp.float32
    part_wv = jax.ShapeDtypeStruct((_N, _DH), f32)
    part_z = jax.ShapeDtypeStruct((_NZ, _DH), f32)
    fn = pl.kernel(
        _full_body,
        out_type=(part_wv, part_z, part_wv, part_z),
        mesh=plsc.VectorSubcoreMesh(core_axis_name="c", subcore_axis_name="s"),
        scratch_types=_sc_scratch(f32, pltpu.VMEM((_C, 16), f32))
        + [pltpu.SemaphoreType.DMA] * 9,
    )
    return fn(q, k, v, aux, fsrc3, fdst3)


def _fin_body(wva0_ref, wva1_ref, za0_ref, za1_ref,
              wvb0_ref, wvb1_ref, zb0_ref, zb1_ref, m_ref, o_ref):
    m = m_ref[...]
    za = za0_ref[...][:, :_H] + za1_ref[...][:, :_H] + 1e-6
    zb = zb0_ref[...][:, :_H] + zb1_ref[...][:, :_H] + 1e-6
    zf = jnp.dot(za, m, preferred_element_type=jnp.float32)
    z2f = jnp.dot(zb, m, preferred_element_type=jnp.float32)
    o_ref[...] = ((wva0_ref[...] + wva1_ref[...]) / zf
                  + (wvb0_ref[...] + wvb1_ref[...]) / z2f)


def _finalize(wva0, wva1, za0, za1, wvb0, wvb1, zb0, zb1, m):
    return pl.pallas_call(
        _fin_body,
        out_shape=jax.ShapeDtypeStruct((_N, _DH), jnp.float32),
    )(wva0, wva1, za0, za1, wvb0, wvb1, zb0, zb1, m)


def kernel(h, e, edge_index, full_edge_index, adj2, rel_pos_3d,
           WQ, bQ, WK, bK, WV, bV, We, be):
    f32 = jnp.float32
    h = h.astype(f32)
    e = e.astype(f32)
    src3 = edge_index[0].astype(jnp.int32).reshape(-1, _SUB, _C)
    dst3 = edge_index[1].astype(jnp.int32).reshape(-1, _SUB, _C)
    fsrc3 = full_edge_index[0].astype(jnp.int32).reshape(-1, _SUB, _C)
    fdst3 = full_edge_index[1].astype(jnp.int32).reshape(-1, _SUB, _C)
    aux = jnp.concatenate(
        [rel_pos_3d.astype(f32), adj2.reshape(-1, 1).astype(f32),
         jnp.zeros((_E, 7), f32)], axis=1)

    wq = (WQ * 0.25).astype(f32)
    b3 = jnp.stack([bQ * 0.25, bK, bV]).astype(f32)

    q, k, v = _qkv(h, wq, WK.astype(f32), WV.astype(f32), b3)
    wvb0, zpb0, wvb1, zpb1 = _sc_full(q, k, v, aux, fsrc3, fdst3)
    pe = _proj_e(e, We.astype(f32), be.reshape(1, _DH).astype(f32))
    eout, wva0, zpa0, wva1, zpa1 = _sc_sparse(q, k, v, pe, src3, dst3)

    za0 = zpa0.reshape(_NPAD, 16)[:_N]
    za1 = zpa1.reshape(_NPAD, 16)[:_N]
    zb0 = zpb0.reshape(_NPAD, 16)[:_N]
    zb1 = zpb1.reshape(_NPAD, 16)[:_N]

    m = (jnp.arange(_DH)[None, :] // _D == jnp.arange(_H)[:, None]).astype(f32)
    h_out = _finalize(wva0, wva1, za0, za1, wvb0, wvb1, zb0, zb1, m)

    return h_out.reshape(_N, _H, _D), eout.reshape(_E, _H, _D)


# speedup vs baseline: 1.6935x; 1.4652x over previous
import jax
import jax.numpy as jnp
from jax import lax
from jax.experimental import pallas as pl
from jax.experimental.pallas import tpu as pltpu
from jax.experimental.pallas import tpu_sc as plsc

_N = 10000
_E = 320000
_H = 8
_D = 16
_DH = _H * _D
_NS = 16
_NW = 32
_C = 16
_SUB = 5
_EPT = _E // _NW
_NPT = 624
_NPTL = _N - 15 * _NPT
_NPAD = 10240
_NZ = _NPAD // 8
_NZT = _NZ // _NS
_NCH = _EPT // (_C * _SUB)


def _qkv_body(h_ref, wq_ref, wk_ref, wv_ref, b3_ref, q_ref, k_ref, v_ref):
    x = h_ref[...]
    q_ref[...] = jnp.dot(x, wq_ref[...], preferred_element_type=jnp.float32) + b3_ref[0:1, :]
    k_ref[...] = jnp.dot(x, wk_ref[...], preferred_element_type=jnp.float32) + b3_ref[1:2, :]
    v_ref[...] = jnp.dot(x, wv_ref[...], preferred_element_type=jnp.float32) + b3_ref[2:3, :]


def _qkv(h, wq, wk, wv, b3):
    out = jax.ShapeDtypeStruct((_N, _DH), jnp.float32)
    return pl.pallas_call(_qkv_body, out_shape=(out, out, out))(h, wq, wk, wv, b3)


_PE_BLK = 4000


def _pe_body(e_ref, w_ref, b_ref, o_ref):
    o_ref[...] = jnp.dot(e_ref[...], w_ref[...], preferred_element_type=jnp.float32) + b_ref[...]


def _proj_e(e, we, be2):
    grid = _E // _PE_BLK
    return pl.pallas_call(
        _pe_body,
        grid=(grid,),
        in_specs=[
            pl.BlockSpec((_PE_BLK, _DH), lambda i: (i, 0)),
            pl.BlockSpec((_DH, _DH), lambda i: (0, 0)),
            pl.BlockSpec((1, _DH), lambda i: (0, 0)),
        ],
        out_specs=pl.BlockSpec((_PE_BLK, _DH), lambda i: (i, 0)),
        out_shape=jax.ShapeDtypeStruct((_E, _DH), jnp.float32),
    )(e, we, be2)


def _lane_total(x, lanes):
    for sh in (8, 4, 2, 1):
        idx = lax.bitwise_xor(lanes, sh)
        x = x + x.at[idx].get(mode="promise_in_bounds")
    return x


def _acc_prologue(sid, v0):
    row0 = sid * _NPT
    nrows16 = jnp.where(sid == _NS - 1, _NPTL // 16, _NPT // 16)
    zrow0 = sid * _NZT

    def _zero_v(r, carry):
        for cix in range(8):
            v0[r, pl.ds(cix * 16, 16)] = jnp.zeros((16,), jnp.float32)
        return carry

    lax.fori_loop(0, _C, _zero_v, 0)
    return row0, nrows16, zrow0


def _acc_writeout(acc_wv, acc_z, wv_hbm, zp_hbm, row0, nrows16, zrow0, buf):
    def _wo(t, carry):
        pltpu.sync_copy(acc_wv.at[pl.ds(row0 + t * 16, 16)], buf)
        pltpu.sync_copy(buf, wv_hbm.at[pl.ds(row0 + t * 16, 16)])
        return carry

    lax.fori_loop(0, nrows16, _wo, 0)
    for jj in range(_NZT // 16):
        pltpu.sync_copy(acc_z.at[pl.ds(zrow0 + jj * 16, 16)], buf)
        pltpu.sync_copy(buf, zp_hbm.at[pl.ds(zrow0 + jj * 16, 16)])


def _make_store_z(didx, lanes):
    def _store_z(zb, i, j, srow_vec):
        dvec = didx[j, :]
        ilane = jnp.full((16,), i, jnp.int32)
        d7 = dvec.at[ilane].get(mode="promise_in_bounds") & 7
        one = jnp.full((16,), 1, jnp.int32)
        for cg in range(8):
            m = (one - jnp.minimum(lax.bitwise_xor(d7, cg), one)).astype(jnp.float32)
            zb[i, pl.ds(cg * 16, 16)] = srow_vec * m

    return _store_z


def _sparse_body(q_hbm, k_hbm, v_hbm, pe_hbm, src_hbm, dst_hbm,
                 eout_hbm, wv0_hbm, zp0_hbm, wv1_hbm, zp1_hbm,
                 sidx, didx, k0, k1, q0, v0, v1, z0, z1, perow, dz0, dz1,
                 acc_wv, acc_z,
                 semk0, semk1, semq, semv0, semv1,
                 semw0, semw1, semz0, semz1, semeo):
    core = lax.axis_index("c")
    sid = lax.axis_index("s")
    wid = sid * 2 + core
    lanes = lax.iota(jnp.int32, 16)
    kb_ = (k0, k1)
    vb_ = (v0, v1)
    zb_ = (z0, z1)
    dz_ = (dz0, dz1)
    semk_ = (semk0, semk1)
    semv_ = (semv0, semv1)
    semw_ = (semw0, semw1)
    semz_ = (semz0, semz1)
    _store_z = _make_store_z(didx, lanes)

    row0, nrows16, zrow0 = _acc_prologue(sid, v0)

    def _zero_acc(t, carry):
        pltpu.sync_copy(v0, acc_wv.at[pl.ds(row0 + t * 16, 16)])
        return carry

    lax.fori_loop(0, nrows16, _zero_acc, 0)
    for jj in range(_NZT // 16):
        pltpu.sync_copy(v0, acc_z.at[pl.ds(zrow0 + jj * 16, 16)])
    plsc.subcore_barrier()

    def _gathers(j, p):
        hk = pltpu.async_copy(k_hbm.at[sidx.at[j]], kb_[p], semk_[p])
        hv = pltpu.async_copy(v_hbm.at[sidx.at[j]], vb_[p], semv_[p])
        return hk, hv

    def _blk(c, carry):
        blk = wid * _NCH + c
        pltpu.sync_copy(src_hbm.at[blk], sidx)
        pltpu.sync_copy(dst_hbm.at[blk], didx)
        gath = {0: _gathers(0, 0)}
        hq = pltpu.async_copy(q_hbm.at[didx.at[0]], q0, semq)
        pend_wv = {}
        pend_z = {}
        pend_eo = {}
        for j in range(_SUB):
            p = j & 1
            kb, vb, zb = kb_[p], vb_[p], zb_[p]
            base = blk * (_C * _SUB) + j * _C
            if j >= 1:
                pend_eo[j - 1].wait()
            pltpu.sync_copy(pe_hbm.at[pl.ds(base, _C)], perow)
            if j + 1 < _SUB:
                if j >= 1:
                    pend_wv[j - 1].wait()
                    pend_z[j - 1].wait()
                gath[j + 1] = _gathers(j + 1, 1 - p)
            dz_[p][...] = lax.shift_right_logical(didx[j, :], 3)
            for hh in gath[j]:
                hh.wait()
            hq.wait()

            @plsc.parallel_loop(0, _C, unroll=1)
            def _edge(i):
                srow_vec = jnp.zeros((16,), jnp.float32)
                for hd in range(_H):
                    sl = pl.ds(hd * 16, 16)
                    eo = kb[i, sl] * q0[i, sl] * perow[i, sl]
                    kb[i, sl] = eo
                    srow_vec = jnp.where(lanes == hd, _lane_total(eo, lanes),
                                         srow_vec)
                sexp = jnp.exp(jnp.clip(srow_vec, -5.0, 5.0))
                sexp = jnp.where(lanes < _H, sexp, 0.0)
                for hd in range(_H):
                    sl = pl.ds(hd * 16, 16)
                    sv = sexp.at[jnp.full((16,), hd, jnp.int32)].get(
                        mode="promise_in_bounds")
                    vb[i, sl] = vb[i, sl] * sv
                _store_z(zb, i, j, sexp)

            if j + 1 < _SUB:
                hq = pltpu.async_copy(q_hbm.at[didx.at[j + 1]], q0, semq)
            pend_eo[j] = pltpu.async_copy(kb, eout_hbm.at[pl.ds(base, _C)], semeo)
            pend_wv[j] = pltpu.async_copy(vb, acc_wv.at[didx.at[j]], semw_[p],
                                          add=True)
            pend_z[j] = pltpu.async_copy(zb, acc_z.at[dz_[p]], semz_[p],
                                         add=True)
        pend_eo[_SUB - 1].wait()
        for j in (_SUB - 2, _SUB - 1):
            pend_wv[j].wait()
            pend_z[j].wait()
        return carry

    lax.fori_loop(0, _NCH, _blk, 0)
    plsc.subcore_barrier()

    @pl.when(core == 0)
    def _():
        _acc_writeout(acc_wv, acc_z, wv0_hbm, zp0_hbm, row0, nrows16, zrow0, v0)

    @pl.when(core == 1)
    def _():
        _acc_writeout(acc_wv, acc_z, wv1_hbm, zp1_hbm, row0, nrows16, zrow0, v0)


def _full_body(q_hbm, k_hbm, v_hbm, aux_hbm, src_hbm, dst_hbm,
               wv0_hbm, zp0_hbm, wv1_hbm, zp1_hbm,
               sidx, didx, k0, k1, q0, v0, v1, z0, z1, auxrow, dz0, dz1,
               acc_wv, acc_z,
               semk0, semk1, semq, semv0, semv1,
               semw0, semw1, semz0, semz1):
    core = lax.axis_index("c")
    sid = lax.axis_index("s")
    wid = sid * 2 + core
    lanes = lax.iota(jnp.int32, 16)
    kb_ = (k0, k1)
    vb_ = (v0, v1)
    zb_ = (z0, z1)
    dz_ = (dz0, dz1)
    semk_ = (semk0, semk1)
    semv_ = (semv0, semv1)
    semw_ = (semw0, semw1)
    semz_ = (semz0, semz1)
    _store_z = _make_store_z(didx, lanes)

    row0, nrows16, zrow0 = _acc_prologue(sid, v0)

    def _zero_acc(t, carry):
        pltpu.sync_copy(v0, acc_wv.at[pl.ds(row0 + t * 16, 16)])
        return carry

    lax.fori_loop(0, nrows16, _zero_acc, 0)
    for jj in range(_NZT // 16):
        pltpu.sync_copy(v0, acc_z.at[pl.ds(zrow0 + jj * 16, 16)])
    plsc.subcore_barrier()

    def _gathers(j, p):
        hk = pltpu.async_copy(k_hbm.at[sidx.at[j]], kb_[p], semk_[p])
        hv = pltpu.async_copy(v_hbm.at[sidx.at[j]], vb_[p], semv_[p])
        return hk, hv

    def _blk(c, carry):
        blk = wid * _NCH + c
        pltpu.sync_copy(src_hbm.at[blk], sidx)
        pltpu.sync_copy(dst_hbm.at[blk], didx)
        gath = {0: _gathers(0, 0)}
        hq = pltpu.async_copy(q_hbm.at[didx.at[0]], q0, semq)
        pend_wv = {}
        pend_z = {}
        for j in range(_SUB):
            p = j & 1
            kb, vb, zb = kb_[p], vb_[p], zb_[p]
            base = blk * (_C * _SUB) + j * _C
            pltpu.sync_copy(aux_hbm.at[pl.ds(base, _C)], auxrow)
            if j + 1 < _SUB:
                if j >= 1:
                    pend_wv[j - 1].wait()
                    pend_z[j - 1].wait()
                gath[j + 1] = _gathers(j + 1, 1 - p)
            dz_[p][...] = lax.shift_right_logical(didx[j, :], 3)
            for hh in gath[j]:
                hh.wait()
            hq.wait()

            @plsc.parallel_loop(0, _C, unroll=1)
            def _edge(i):
                srow_vec = jnp.zeros((16,), jnp.float32)
                auxvec = auxrow[i, :]
                av = auxvec[8]
                for hd in range(_H):
                    sl = pl.ds(hd * 16, 16)
                    tot = _lane_total(kb[i, sl] * q0[i, sl], lanes)
                    srow_vec = jnp.where(lanes == hd, tot + auxvec[hd],
                                         srow_vec)
                sexp = jnp.exp(jnp.clip(srow_vec, -5.0, 5.0) * av)
                sexp = jnp.where(lanes < _H, sexp, 0.0)
                for hd in range(_H):
                    sl = pl.ds(hd * 16, 16)
                    sv = sexp.at[jnp.full((16,), hd, jnp.int32)].get(
                        mode="promise_in_bounds")
                    vb[i, sl] = vb[i, sl] * sv
                _store_z(zb, i, j, sexp)

            if j + 1 < _SUB:
                hq = pltpu.async_copy(q_hbm.at[didx.at[j + 1]], q0, semq)
            pend_wv[j] = pltpu.async_copy(vb, acc_wv.at[didx.at[j]], semw_[p],
                                          add=True)
            pend_z[j] = pltpu.async_copy(zb, acc_z.at[dz_[p]], semz_[p],
                                         add=True)
        for j in (_SUB - 2, _SUB - 1):
            pend_wv[j].wait()
            pend_z[j].wait()
        return carry

    lax.fori_loop(0, _NCH, _blk, 0)
    plsc.subcore_barrier()

    @pl.when(core == 0)
    def _():
        _acc_writeout(acc_wv, acc_z, wv0_hbm, zp0_hbm, row0, nrows16, zrow0, v0)

    @pl.when(core == 1)
    def _():
        _acc_writeout(acc_wv, acc_z, wv1_hbm, zp1_hbm, row0, nrows16, zrow0, v0)


def _sc_scratch(f32, extra):
    return [
        pltpu.VMEM((_SUB, _C), jnp.int32),
        pltpu.VMEM((_SUB, _C), jnp.int32),
        pltpu.VMEM((_C, _DH), f32),
        pltpu.VMEM((_C, _DH), f32),
        pltpu.VMEM((_C, _DH), f32),
        pltpu.VMEM((_C, _DH), f32),
        pltpu.VMEM((_C, _DH), f32),
        pltpu.VMEM((_C, _DH), f32),
        pltpu.VMEM((_C, _DH), f32),
        extra,
        pltpu.VMEM((_C,), jnp.int32),
        pltpu.VMEM((_C,), jnp.int32),
        pltpu.VMEM_SHARED((_N, _DH), f32),
        pltpu.VMEM_SHARED((_NZ, _DH), f32),
    ]


def _sc_sparse(q, k, v, pe, src3, dst3):
    f32 = jnp.float32
    part_wv = jax.ShapeDtypeStruct((_N, _DH), f32)
    part_z = jax.ShapeDtypeStruct((_NZ, _DH), f32)
    fn = pl.kernel(
        _sparse_body,
        out_type=(jax.ShapeDtypeStruct((_E, _DH), f32),
                  part_wv, part_z, part_wv, part_z),
        mesh=plsc.VectorSubcoreMesh(core_axis_name="c", subcore_axis_name="s"),
        scratch_types=_sc_scratch(f32, pltpu.VMEM((_C, _DH), f32))
        + [pltpu.SemaphoreType.DMA] * 10,
    )
    return fn(q, k, v, pe, src3, dst3)


def _sc_full(q, k, v, aux, fsrc3, fdst3):
    f32 = jnp.float32
    part_wv = jax.ShapeDtypeStruct((_N, _DH), f32)
    part_z = jax.ShapeDtypeStruct((_NZ, _DH), f32)
    fn = pl.kernel(
        _full_body,
        out_type=(part_wv, part_z, part_wv, part_z),
        mesh=plsc.VectorSubcoreMesh(core_axis_name="c", subcore_axis_name="s"),
        scratch_types=_sc_scratch(f32, pltpu.VMEM((_C, 16), f32))
        + [pltpu.SemaphoreType.DMA] * 9,
    )
    return fn(q, k, v, aux, fsrc3, fdst3)


def _fin_body(wva0_ref, wva1_ref, za0_ref, za1_ref,
              wvb0_ref, wvb1_ref, zb0_ref, zb1_ref, m_ref, o_ref):
    m = m_ref[...]
    za = za0_ref[...][:, :_H] + za1_ref[...][:, :_H] + 1e-6
    zb = zb0_ref[...][:, :_H] + zb1_ref[...][:, :_H] + 1e-6
    zf = jnp.dot(za, m, preferred_element_type=jnp.float32)
    z2f = jnp.dot(zb, m, preferred_element_type=jnp.float32)
    o_ref[...] = ((wva0_ref[...] + wva1_ref[...]) / zf
                  + (wvb0_ref[...] + wvb1_ref[...]) / z2f)


def _finalize(wva0, wva1, za0, za1, wvb0, wvb1, zb0, zb1, m):
    return pl.pallas_call(
        _fin_body,
        out_shape=jax.ShapeDtypeStruct((_N, _DH), jnp.float32),
    )(wva0, wva1, za0, za1, wvb0, wvb1, zb0, zb1, m)


def kernel(h, e, edge_index, full_edge_index, adj2, rel_pos_3d,
           WQ, bQ, WK, bK, WV, bV, We, be):
    f32 = jnp.float32
    h = h.astype(f32)
    e = e.astype(f32)
    src3 = edge_index[0].astype(jnp.int32).reshape(-1, _SUB, _C)
    dst3 = edge_index[1].astype(jnp.int32).reshape(-1, _SUB, _C)
    fsrc3 = full_edge_index[0].astype(jnp.int32).reshape(-1, _SUB, _C)
    fdst3 = full_edge_index[1].astype(jnp.int32).reshape(-1, _SUB, _C)
    aux = jnp.concatenate(
        [rel_pos_3d.astype(f32), adj2.reshape(-1, 1).astype(f32),
         jnp.zeros((_E, 7), f32)], axis=1)

    wq = (WQ * 0.25).astype(f32)
    b3 = jnp.stack([bQ * 0.25, bK, bV]).astype(f32)

    q, k, v = _qkv(h, wq, WK.astype(f32), WV.astype(f32), b3)
    wvb0, zpb0, wvb1, zpb1 = _sc_full(q, k, v, aux, fsrc3, fdst3)
    pe = _proj_e(e, We.astype(f32), be.reshape(1, _DH).astype(f32))
    eout, wva0, zpa0, wva1, zpa1 = _sc_sparse(q, k, v, pe, src3, dst3)

    za0 = zpa0.reshape(_NPAD, 16)[:_N]
    za1 = zpa1.reshape(_NPAD, 16)[:_N]
    zb0 = zpb0.reshape(_NPAD, 16)[:_N]
    zb1 = zpb1.reshape(_NPAD, 16)[:_N]

    m = (jnp.arange(_DH)[None, :] // _D == jnp.arange(_H)[:, None]).astype(f32)
    h_out = _finalize(wva0, wva1, za0, za1, wvb0, wvb1, zb0, zb1, m)

    return h_out.reshape(_N, _H, _D), eout.reshape(_E, _H, _D)
